# baseline (speedup 1.0000x reference)
import functools

import jax
import jax.numpy as jnp
from jax import lax
from jax.experimental import pallas as pl
from jax.experimental.pallas import tpu as pltpu
from jax.experimental.pallas import tpu_sc as plsc

DIM = 128
CHUNK = 128
NW = 32


def _make_sc_gather(num_rows_table, total_rows):
    rows_per_w = total_rows // NW
    nchunks = rows_per_w // CHUNK
    mesh = plsc.VectorSubcoreMesh(core_axis_name="c", subcore_axis_name="s")

    nchunks_pad = ((nchunks + 7) // 8) * 8

    @functools.partial(
        pl.kernel,
        out_type=jax.ShapeDtypeStruct((total_rows, DIM), jnp.float32),
        mesh=mesh,
        scratch_types=[
            pltpu.VMEM((nchunks_pad, CHUNK), jnp.int32),
            pltpu.VMEM((CHUNK, DIM), jnp.float32),
            pltpu.VMEM((CHUNK, DIM), jnp.float32),
            pltpu.SemaphoreType.DMA,
            pltpu.SemaphoreType.DMA,
        ],
    )
    def gather_k(emb_hbm, nodes_hbm, out_hbm, idx_v, buf0, buf1, sem0, sem1):
        wid = lax.axis_index("s") * 2 + lax.axis_index("c")
        base_row = wid * rows_per_w
        pltpu.sync_copy(nodes_hbm.at[wid], idx_v)

        @pl.loop(0, nchunks, step=2)
        def _(j0):
            j1 = j0 + 1
            c0 = pltpu.async_copy(emb_hbm.at[idx_v.at[j0]], buf0, sem0)
            c1 = pltpu.async_copy(emb_hbm.at[idx_v.at[j1]], buf1, sem1)
            c0.wait()
            pltpu.sync_copy(buf0, out_hbm.at[pl.ds(base_row + j0 * CHUNK, CHUNK)])
            c1.wait()
            pltpu.sync_copy(buf1, out_hbm.at[pl.ds(base_row + j1 * CHUNK, CHUNK)])

    return gather_k


BS = 16


def _tc_body(adj_ref, g_ref, npm_ref, tm_ref, w0_ref, pw2_ref, t1_ref,
             gw_ref, b_ref, out_ref, hs_ref):
    f32 = jnp.float32
    bf16 = jnp.bfloat16
    w0 = w0_ref[...]
    pw2 = pw2_ref[...]
    t1 = t1_ref[...][:4]
    gw = gw_ref[...]
    bias = b_ref[...]
    N = adj_ref.shape[1]

    def clamp_bcast(i):
        tmc = tm_ref[i][:, :1]
        clamp = jnp.minimum(tmc, 1).astype(f32)
        return jnp.broadcast_to(clamp, (N, DIM))

    for i in range(BS):
        npm = npm_ref[i]
        tmc = tm_ref[i][:, :1]
        clamp = jnp.minimum(tmc, 1).astype(f32)
        pos_num = jnp.sum(npm.astype(f32), axis=1, keepdims=True)
        scale = clamp / (pos_num + 1e-9)
        oneh = (tmc == lax.broadcasted_iota(jnp.int32, (N, 4), 1)).astype(f32)
        h = jnp.dot(g_ref[i], w0, preferred_element_type=f32)
        h = h + jnp.dot(npm, pw2, preferred_element_type=f32) * scale
        h = h + jnp.dot(oneh, t1, preferred_element_type=f32)
        hs_ref[i] = h.astype(bf16)

    for layer in range(2):
        for i in range(BS):
            t = jnp.dot(adj_ref[i], hs_ref[i], preferred_element_type=f32)
            h = jnp.maximum(
                jnp.dot(t, gw, preferred_element_type=f32) + bias, 0.0)
            h = h * clamp_bcast(i)
            if layer == 0:
                hs_ref[i] = h.astype(bf16)
            else:
                out_ref[i] = h


def kernel(adj, nodes, node_type_mask, node_pos_matrix, emb, pos_emb, type_emb,
           w_pos_type, gcn_w, gcn_b):
    B, N, _ = adj.shape
    L = node_pos_matrix.shape[-1]
    total_rows = B * N

    nchunks = total_rows // CHUNK // NW
    nchunks_pad = ((nchunks + 7) // 8) * 8
    nodes_3d = nodes.astype(jnp.int32).reshape(NW, nchunks, CHUNK)
    nodes_3d = jnp.pad(nodes_3d, ((0, 0), (0, nchunks_pad - nchunks), (0, 0)))
    g = _make_sc_gather(emb.shape[0], total_rows)(emb, nodes_3d)
    g = g.reshape(B, N, DIM)

    adj_bf = adj.astype(jnp.bfloat16)
    tm8 = jnp.broadcast_to(node_type_mask.astype(jnp.int32)[:, :, None],
                           (B, N, 8))
    npm_bf = node_pos_matrix.astype(jnp.bfloat16)
    t1 = jnp.pad(jnp.dot(type_emb, w_pos_type[DIM:2 * DIM]), ((0, 4), (0, 0)))
    pw2 = jnp.dot(pos_emb[:L], w_pos_type[2 * DIM:]).astype(jnp.bfloat16)
    w0 = w_pos_type[:DIM]
    b2 = gcn_b.reshape(1, DIM)

    out = pl.pallas_call(
        _tc_body,
        grid=(B // BS,),
        in_specs=[
            pl.BlockSpec((BS, N, N), lambda b: (b, 0, 0)),
            pl.BlockSpec((BS, N, DIM), lambda b: (b, 0, 0)),
            pl.BlockSpec((BS, N, L), lambda b: (b, 0, 0)),
            pl.BlockSpec((BS, N, 8), lambda b: (b, 0, 0)),
            pl.BlockSpec((DIM, DIM), lambda b: (0, 0)),
            pl.BlockSpec((L, DIM), lambda b: (0, 0)),
            pl.BlockSpec((8, DIM), lambda b: (0, 0)),
            pl.BlockSpec((DIM, DIM), lambda b: (0, 0)),
            pl.BlockSpec((1, DIM), lambda b: (0, 0)),
        ],
        out_specs=pl.BlockSpec((BS, N, DIM), lambda b: (b, 0, 0)),
        out_shape=jax.ShapeDtypeStruct((B, N, DIM), jnp.float32),
        scratch_shapes=[pltpu.VMEM((BS, N, DIM), jnp.bfloat16)],
    )(adj_bf, g, npm_bf, tm8, w0, pw2, t1, gcn_w, b2)
    return out


# speedup vs baseline: 1.1479x; 1.1479x over previous
import functools

import jax
import jax.numpy as jnp
from jax import lax
from jax.experimental import pallas as pl
from jax.experimental.pallas import tpu as pltpu
from jax.experimental.pallas import tpu_sc as plsc

DIM = 128
CHUNK = 128
NW = 32


def _make_sc_gather(num_rows_table, total_rows):
    rows_per_w = total_rows // NW
    nchunks = rows_per_w // CHUNK
    mesh = plsc.VectorSubcoreMesh(core_axis_name="c", subcore_axis_name="s")

    nchunks_pad = ((nchunks + 7) // 8) * 8

    @functools.partial(
        pl.kernel,
        out_type=jax.ShapeDtypeStruct((total_rows, DIM), jnp.float32),
        mesh=mesh,
        scratch_types=[
            pltpu.VMEM((nchunks_pad, CHUNK), jnp.int32),
            pltpu.VMEM((CHUNK, DIM), jnp.float32),
            pltpu.VMEM((CHUNK, DIM), jnp.float32),
            pltpu.SemaphoreType.DMA,
            pltpu.SemaphoreType.DMA,
        ],
    )
    def gather_k(emb_hbm, nodes_hbm, out_hbm, idx_v, buf0, buf1, sem0, sem1):
        wid = lax.axis_index("s") * 2 + lax.axis_index("c")
        base_row = wid * rows_per_w
        pltpu.sync_copy(nodes_hbm.at[wid], idx_v)

        @pl.loop(0, nchunks, step=2)
        def _(j0):
            j1 = j0 + 1
            c0 = pltpu.async_copy(emb_hbm.at[idx_v.at[j0]], buf0, sem0)
            c1 = pltpu.async_copy(emb_hbm.at[idx_v.at[j1]], buf1, sem1)
            c0.wait()
            pltpu.sync_copy(buf0, out_hbm.at[pl.ds(base_row + j0 * CHUNK, CHUNK)])
            c1.wait()
            pltpu.sync_copy(buf1, out_hbm.at[pl.ds(base_row + j1 * CHUNK, CHUNK)])

    return gather_k


BS = 16


def _tc_body(adj_ref, g_ref, npm_ref, tm_ref, wcat_ref, gw_ref, b_ref,
             out_ref):
    f32 = jnp.float32
    wcat = wcat_ref[...]
    gw = gw_ref[...]
    bias = b_ref[...]
    N = adj_ref.shape[1]

    def clamp_bcast(i):
        tmc = tm_ref[i][:, :1]
        clamp = jnp.minimum(tmc, 1).astype(f32)
        return jnp.broadcast_to(clamp, (N, DIM))

    for i in range(BS):
        gb = g_ref[i]
        npm = npm_ref[i]
        tmc = tm_ref[i][:, :1]
        clamp = jnp.minimum(tmc, 1).astype(f32)
        pos_num = jnp.sum(npm, axis=1, keepdims=True)
        scale = clamp / (pos_num + 1e-9)
        oneh = (tmc == lax.broadcasted_iota(jnp.int32, (N, 4), 1)).astype(f32)
        lhs = jnp.concatenate([gb, npm * scale, oneh], axis=1)
        out_ref[i] = jnp.dot(lhs, wcat, preferred_element_type=f32)

    for _ in range(2):
        for i in range(BS):
            t = jnp.dot(adj_ref[i], out_ref[i], preferred_element_type=f32)
            h = jnp.maximum(
                jnp.dot(t, gw, preferred_element_type=f32) + bias, 0.0)
            out_ref[i] = h * clamp_bcast(i)


def kernel(adj, nodes, node_type_mask, node_pos_matrix, emb, pos_emb, type_emb,
           w_pos_type, gcn_w, gcn_b):
    B, N, _ = adj.shape
    L = node_pos_matrix.shape[-1]
    total_rows = B * N

    nchunks = total_rows // CHUNK // NW
    nchunks_pad = ((nchunks + 7) // 8) * 8
    nodes_3d = nodes.astype(jnp.int32).reshape(NW, nchunks, CHUNK)
    nodes_3d = jnp.pad(nodes_3d, ((0, 0), (0, nchunks_pad - nchunks), (0, 0)))
    g = _make_sc_gather(emb.shape[0], total_rows)(emb, nodes_3d)
    g = g.reshape(B, N, DIM)

    tm8 = jnp.broadcast_to(node_type_mask.astype(jnp.int32)[:, :, None],
                           (B, N, 8))
    t1 = jnp.dot(type_emb, w_pos_type[DIM:2 * DIM])
    pw2 = jnp.dot(pos_emb[:L], w_pos_type[2 * DIM:])
    wcat = jnp.concatenate([w_pos_type[:DIM], pw2, t1], axis=0)
    b2 = gcn_b.reshape(1, DIM)

    out = pl.pallas_call(
        _tc_body,
        grid=(B // BS,),
        in_specs=[
            pl.BlockSpec((BS, N, N), lambda b: (b, 0, 0)),
            pl.BlockSpec((BS, N, DIM), lambda b: (b, 0, 0)),
            pl.BlockSpec((BS, N, L), lambda b: (b, 0, 0)),
            pl.BlockSpec((BS, N, 8), lambda b: (b, 0, 0)),
            pl.BlockSpec((DIM + L + 4, DIM), lambda b: (0, 0)),
            pl.BlockSpec((DIM, DIM), lambda b: (0, 0)),
            pl.BlockSpec((1, DIM), lambda b: (0, 0)),
        ],
        out_specs=pl.BlockSpec((BS, N, DIM), lambda b: (b, 0, 0)),
        out_shape=jax.ShapeDtypeStruct((B, N, DIM), jnp.float32),
    )(adj, g, node_pos_matrix, tm8, wcat, gcn_w, b2)
    return out


# speedup vs baseline: 1.1768x; 1.0251x over previous
import functools

import jax
import jax.numpy as jnp
from jax import lax
from jax.experimental import pallas as pl
from jax.experimental.pallas import tpu as pltpu
from jax.experimental.pallas import tpu_sc as plsc

DIM = 128
CHUNK = 128
NW = 32


def _make_sc_gather(num_rows_table, total_rows):
    rows_per_w = total_rows // NW
    nchunks = rows_per_w // CHUNK
    mesh = plsc.VectorSubcoreMesh(core_axis_name="c", subcore_axis_name="s")

    nchunks_pad = ((nchunks + 7) // 8) * 8

    @functools.partial(
        pl.kernel,
        out_type=jax.ShapeDtypeStruct((total_rows, DIM), jnp.float32),
        mesh=mesh,
        scratch_types=[
            pltpu.VMEM((nchunks_pad, CHUNK), jnp.int32),
            pltpu.VMEM((CHUNK, DIM), jnp.float32),
            pltpu.VMEM((CHUNK, DIM), jnp.float32),
            pltpu.SemaphoreType.DMA,
            pltpu.SemaphoreType.DMA,
        ],
    )
    def gather_k(emb_hbm, nodes_hbm, out_hbm, idx_v, buf0, buf1, sem0, sem1):
        wid = lax.axis_index("s") * 2 + lax.axis_index("c")
        base_row = wid * rows_per_w
        pltpu.sync_copy(nodes_hbm.at[wid], idx_v)

        @pl.loop(0, nchunks, step=2)
        def _(j0):
            j1 = j0 + 1
            c0 = pltpu.async_copy(emb_hbm.at[idx_v.at[j0]], buf0, sem0)
            c1 = pltpu.async_copy(emb_hbm.at[idx_v.at[j1]], buf1, sem1)
            c0.wait()
            pltpu.sync_copy(buf0, out_hbm.at[pl.ds(base_row + j0 * CHUNK, CHUNK)])
            c1.wait()
            pltpu.sync_copy(buf1, out_hbm.at[pl.ds(base_row + j1 * CHUNK, CHUNK)])

    return gather_k


BS = 32


def _tc_body(adj_ref, g_ref, npm_ref, tm_ref, wcat_ref, gw_ref, b_ref,
             out_ref):
    f32 = jnp.float32
    wcat = wcat_ref[...]
    gw = gw_ref[...]
    bias = b_ref[...]
    N = adj_ref.shape[1]

    def clamp_bcast(i):
        tmc = tm_ref[i][:, :1]
        clamp = jnp.minimum(tmc, 1).astype(f32)
        return jnp.broadcast_to(clamp, (N, DIM))

    for i in range(BS):
        gb = g_ref[i]
        npm = npm_ref[i]
        tmc = tm_ref[i][:, :1]
        clamp = jnp.minimum(tmc, 1).astype(f32)
        pos_num = jnp.sum(npm, axis=1, keepdims=True)
        scale = clamp / (pos_num + 1e-9)
        oneh = (tmc == lax.broadcasted_iota(jnp.int32, (N, 4), 1)).astype(f32)
        lhs = jnp.concatenate([gb, npm * scale, oneh], axis=1)
        out_ref[i] = jnp.dot(lhs, wcat, preferred_element_type=f32)

    for _ in range(2):
        for i in range(BS):
            t = jnp.dot(adj_ref[i], out_ref[i], preferred_element_type=f32)
            h = jnp.maximum(
                jnp.dot(t, gw, preferred_element_type=f32) + bias, 0.0)
            out_ref[i] = h * clamp_bcast(i)


def kernel(adj, nodes, node_type_mask, node_pos_matrix, emb, pos_emb, type_emb,
           w_pos_type, gcn_w, gcn_b):
    B, N, _ = adj.shape
    L = node_pos_matrix.shape[-1]
    total_rows = B * N

    nchunks = total_rows // CHUNK // NW
    nchunks_pad = ((nchunks + 7) // 8) * 8
    nodes_3d = nodes.astype(jnp.int32).reshape(NW, nchunks, CHUNK)
    nodes_3d = jnp.pad(nodes_3d, ((0, 0), (0, nchunks_pad - nchunks), (0, 0)))
    g = _make_sc_gather(emb.shape[0], total_rows)(emb, nodes_3d)
    g = g.reshape(B, N, DIM)

    tm8 = jnp.broadcast_to(node_type_mask.astype(jnp.int32)[:, :, None],
                           (B, N, 8))
    t1 = jnp.dot(type_emb, w_pos_type[DIM:2 * DIM])
    pw2 = jnp.dot(pos_emb[:L], w_pos_type[2 * DIM:])
    wcat = jnp.concatenate([w_pos_type[:DIM], pw2, t1], axis=0)
    b2 = gcn_b.reshape(1, DIM)

    out = pl.pallas_call(
        _tc_body,
        grid=(B // BS,),
        in_specs=[
            pl.BlockSpec((BS, N, N), lambda b: (b, 0, 0)),
            pl.BlockSpec((BS, N, DIM), lambda b: (b, 0, 0)),
            pl.BlockSpec((BS, N, L), lambda b: (b, 0, 0)),
            pl.BlockSpec((BS, N, 8), lambda b: (b, 0, 0)),
            pl.BlockSpec((DIM + L + 4, DIM), lambda b: (0, 0)),
            pl.BlockSpec((DIM, DIM), lambda b: (0, 0)),
            pl.BlockSpec((1, DIM), lambda b: (0, 0)),
        ],
        out_specs=pl.BlockSpec((BS, N, DIM), lambda b: (b, 0, 0)),
        out_shape=jax.ShapeDtypeStruct((B, N, DIM), jnp.float32),
    )(adj, g, node_pos_matrix, tm8, wcat, gcn_w, b2)
    return out
